# Initial kernel scaffold; baseline (speedup 1.0000x reference)
import jax
import jax.numpy as jnp
from jax.experimental import pallas as pl


def kernel(t, v, edge_index, w_edge, tau, stimulus, bias):
    raise NotImplementedError("write your pallas kernel here")


# speedup vs baseline: 186.4853x; 186.4853x over previous
import functools

import jax
import jax.numpy as jnp
from jax import lax
from jax.experimental import pallas as pl
from jax.experimental.pallas import tpu as pltpu
from jax.experimental.pallas import tpu_sc as plsc

N = 100000
E = 6400000
CLAMP = 10.0

NSC = 2
NTILE = 16
NWORK = NSC * NTILE
NPAD = 100352
SLICE1 = NPAD // NTILE
SLICE2 = NPAD // NWORK
EPT = E // NWORK
W = 8000
NWIN = EPT // W

_mesh = plsc.VectorSubcoreMesh(core_axis_name="c", subcore_axis_name="s")


@functools.partial(
    pl.kernel,
    out_type=jax.ShapeDtypeStruct((NSC * NPAD,), jnp.float32),
    mesh=_mesh,
    scratch_types=[
        pltpu.VMEM((W,), jnp.int32),
        pltpu.VMEM((W,), jnp.int32),
        pltpu.VMEM((W,), jnp.float32),
        pltpu.VMEM((W,), jnp.float32),
        pltpu.VMEM((SLICE1,), jnp.float32),
        pltpu.VMEM_SHARED((NPAD,), jnp.float32),
        pltpu.VMEM_SHARED((NPAD,), jnp.float32),
        pltpu.SemaphoreType.DMA,
    ],
)
def _scatter_phase(v_hbm, src_hbm, dst_hbm, w_hbm, out_hbm,
                   srcbuf, dstbuf, wbuf, valbuf, slicebuf, vsh, acc, gsem):
    cid = lax.axis_index("c")
    sid = lax.axis_index("s")
    gwid = cid * NTILE + sid
    nbase = sid * SLICE1

    def _zero(j, carry):
        slicebuf[pl.ds(j * 16, 16)] = jnp.zeros((16,), jnp.float32)
        return carry
    lax.fori_loop(0, SLICE1 // 16, _zero, 0)
    pltpu.sync_copy(slicebuf, acc.at[pl.ds(nbase, SLICE1)])

    pltpu.sync_copy(v_hbm.at[pl.ds(nbase, SLICE1)], slicebuf)
    pltpu.sync_copy(slicebuf, vsh.at[pl.ds(nbase, SLICE1)])
    plsc.subcore_barrier()

    ebase = gwid * EPT

    def _window(k, carry):
        off = ebase + k * W
        pltpu.sync_copy(src_hbm.at[pl.ds(off, W)], srcbuf)
        pltpu.sync_copy(dst_hbm.at[pl.ds(off, W)], dstbuf)
        pltpu.sync_copy(w_hbm.at[pl.ds(off, W)], wbuf)
        pltpu.async_copy(vsh.at[srcbuf], valbuf, gsem).wait()

        def _compute(j, c):
            sl = pl.ds(j * 16, 16)
            x = valbuf[sl]
            m = jnp.minimum(jnp.maximum(x, 0.0), CLAMP) * wbuf[sl]
            valbuf[sl] = m
            return c
        lax.fori_loop(0, W // 16, _compute, 0)

        pltpu.sync_copy(valbuf, acc.at[dstbuf], add=True)
        return carry

    lax.fori_loop(0, NWIN, _window, 0)
    plsc.subcore_barrier()

    pltpu.sync_copy(acc.at[pl.ds(nbase, SLICE1)], slicebuf)
    pltpu.sync_copy(slicebuf, out_hbm.at[pl.ds(cid * NPAD + nbase, SLICE1)])


@functools.partial(
    pl.kernel,
    out_type=jax.ShapeDtypeStruct((NPAD,), jnp.float32),
    mesh=_mesh,
    scratch_types=[
        pltpu.VMEM((SLICE2,), jnp.float32),
        pltpu.VMEM((SLICE2,), jnp.float32),
        pltpu.VMEM((SLICE2,), jnp.float32),
        pltpu.VMEM((SLICE2,), jnp.float32),
        pltpu.VMEM((SLICE2,), jnp.float32),
        pltpu.VMEM((SLICE2,), jnp.float32),
    ],
)
def _combine_phase(part_hbm, v_hbm, tau_hbm, stim_hbm, bias_hbm, out_hbm,
                   p0, p1, vb, tb, sb, bb):
    cid = lax.axis_index("c")
    sid = lax.axis_index("s")
    nb = (cid * NTILE + sid) * SLICE2
    pltpu.sync_copy(part_hbm.at[pl.ds(nb, SLICE2)], p0)
    pltpu.sync_copy(part_hbm.at[pl.ds(NPAD + nb, SLICE2)], p1)
    pltpu.sync_copy(v_hbm.at[pl.ds(nb, SLICE2)], vb)
    pltpu.sync_copy(tau_hbm.at[pl.ds(nb, SLICE2)], tb)
    pltpu.sync_copy(stim_hbm.at[pl.ds(nb, SLICE2)], sb)
    pltpu.sync_copy(bias_hbm.at[pl.ds(nb, SLICE2)], bb)

    def _compute(j, carry):
        sl = pl.ds(j * 16, 16)
        vc = jnp.minimum(jnp.maximum(vb[sl], -CLAMP), CLAMP)
        p0[sl] = (p0[sl] + p1[sl] + sb[sl] + bb[sl] - vc) / tb[sl]
        return carry
    lax.fori_loop(0, SLICE2 // 16, _compute, 0)
    pltpu.sync_copy(p0, out_hbm.at[pl.ds(nb, SLICE2)])


def kernel(t, v, edge_index, w_edge, tau, stimulus, bias):
    pad = NPAD - N
    vp = jnp.pad(v, (0, pad))
    taup = jnp.pad(tau, (0, pad), constant_values=1.0)
    stimp = jnp.pad(stimulus, (0, pad))
    biasp = jnp.pad(bias, (0, pad))
    src = edge_index[0]
    dst = edge_index[1]
    partial = _scatter_phase(vp, src, dst, w_edge)
    dvp = _combine_phase(partial, vp, taup, stimp, biasp)
    return dvp[:N]


# speedup vs baseline: 206.2635x; 1.1061x over previous
import functools

import jax
import jax.numpy as jnp
from jax import lax
from jax.experimental import pallas as pl
from jax.experimental.pallas import tpu as pltpu
from jax.experimental.pallas import tpu_sc as plsc

N = 100000
E = 6400000
CLAMP = 10.0

NSC = 2
NTILE = 16
NWORK = NSC * NTILE
NPAD = 100352
SLICE1 = NPAD // NTILE
SLICE2 = NPAD // NWORK
EPT = E // NWORK
W = 8000
NWIN = EPT // W

_mesh = plsc.VectorSubcoreMesh(core_axis_name="c", subcore_axis_name="s")


@functools.partial(
    pl.kernel,
    out_type=jax.ShapeDtypeStruct((NSC * NPAD,), jnp.float32),
    mesh=_mesh,
    scratch_types=[
        pltpu.VMEM((W,), jnp.int32),
        pltpu.VMEM((W,), jnp.int32),
        pltpu.VMEM((W,), jnp.float32),
        pltpu.VMEM((W,), jnp.float32),
        pltpu.VMEM((SLICE1,), jnp.float32),
        pltpu.VMEM_SHARED((NPAD,), jnp.float32),
        pltpu.VMEM_SHARED((NPAD,), jnp.float32),
        pltpu.SemaphoreType.DMA,
    ],
)
def _scatter_phase(v_hbm, src_hbm, dst_hbm, w_hbm, out_hbm,
                   srcbuf, dstbuf, wbuf, valbuf, slicebuf, vsh, acc, gsem):
    cid = lax.axis_index("c")
    sid = lax.axis_index("s")
    gwid = cid * NTILE + sid
    nbase = sid * SLICE1

    def _zero(j, carry):
        slicebuf[pl.ds(j * 16, 16)] = jnp.zeros((16,), jnp.float32)
        return carry
    lax.fori_loop(0, SLICE1 // 16, _zero, 0)
    pltpu.sync_copy(slicebuf, acc.at[pl.ds(nbase, SLICE1)])

    pltpu.sync_copy(v_hbm.at[pl.ds(nbase, SLICE1)], slicebuf)
    pltpu.sync_copy(slicebuf, vsh.at[pl.ds(nbase, SLICE1)])
    plsc.subcore_barrier()

    ebase = gwid * EPT

    def _window(k, carry):
        off = ebase + k * W
        pltpu.sync_copy(src_hbm.at[pl.ds(off, W)], srcbuf)
        pltpu.sync_copy(dst_hbm.at[pl.ds(off, W)], dstbuf)
        pltpu.sync_copy(w_hbm.at[pl.ds(off, W)], wbuf)
        pltpu.async_copy(vsh.at[srcbuf], valbuf, gsem).wait()

        def _compute(j, c):
            sl = pl.ds(j * 16, 16)
            x = valbuf[sl]
            m = jnp.minimum(jnp.maximum(x, 0.0), CLAMP) * wbuf[sl]
            valbuf[sl] = m
            return c
        lax.fori_loop(0, W // 16, _compute, 0, unroll=8)

        pltpu.sync_copy(valbuf, acc.at[dstbuf], add=True)
        return carry

    lax.fori_loop(0, NWIN, _window, 0)
    plsc.subcore_barrier()

    pltpu.sync_copy(acc.at[pl.ds(nbase, SLICE1)], slicebuf)
    pltpu.sync_copy(slicebuf, out_hbm.at[pl.ds(cid * NPAD + nbase, SLICE1)])


@functools.partial(
    pl.kernel,
    out_type=jax.ShapeDtypeStruct((NPAD,), jnp.float32),
    mesh=_mesh,
    scratch_types=[
        pltpu.VMEM((SLICE2,), jnp.float32),
        pltpu.VMEM((SLICE2,), jnp.float32),
        pltpu.VMEM((SLICE2,), jnp.float32),
        pltpu.VMEM((SLICE2,), jnp.float32),
        pltpu.VMEM((SLICE2,), jnp.float32),
        pltpu.VMEM((SLICE2,), jnp.float32),
    ],
)
def _combine_phase(part_hbm, v_hbm, tau_hbm, stim_hbm, bias_hbm, out_hbm,
                   p0, p1, vb, tb, sb, bb):
    cid = lax.axis_index("c")
    sid = lax.axis_index("s")
    nb = (cid * NTILE + sid) * SLICE2
    pltpu.sync_copy(part_hbm.at[pl.ds(nb, SLICE2)], p0)
    pltpu.sync_copy(part_hbm.at[pl.ds(NPAD + nb, SLICE2)], p1)
    pltpu.sync_copy(v_hbm.at[pl.ds(nb, SLICE2)], vb)
    pltpu.sync_copy(tau_hbm.at[pl.ds(nb, SLICE2)], tb)
    pltpu.sync_copy(stim_hbm.at[pl.ds(nb, SLICE2)], sb)
    pltpu.sync_copy(bias_hbm.at[pl.ds(nb, SLICE2)], bb)

    def _compute(j, carry):
        sl = pl.ds(j * 16, 16)
        vc = jnp.minimum(jnp.maximum(vb[sl], -CLAMP), CLAMP)
        p0[sl] = (p0[sl] + p1[sl] + sb[sl] + bb[sl] - vc) / tb[sl]
        return carry
    lax.fori_loop(0, SLICE2 // 16, _compute, 0)
    pltpu.sync_copy(p0, out_hbm.at[pl.ds(nb, SLICE2)])


def kernel(t, v, edge_index, w_edge, tau, stimulus, bias):
    pad = NPAD - N
    vp = jnp.pad(v, (0, pad))
    taup = jnp.pad(tau, (0, pad), constant_values=1.0)
    stimp = jnp.pad(stimulus, (0, pad))
    biasp = jnp.pad(bias, (0, pad))
    src = edge_index[0]
    dst = edge_index[1]
    partial = _scatter_phase(vp, src, dst, w_edge)
    dvp = _combine_phase(partial, vp, taup, stimp, biasp)
    return dvp[:N]


# speedup vs baseline: 211.7015x; 1.0264x over previous
import functools

import jax
import jax.numpy as jnp
from jax import lax
from jax.experimental import pallas as pl
from jax.experimental.pallas import tpu as pltpu
from jax.experimental.pallas import tpu_sc as plsc

N = 100000
E = 6400000
CLAMP = 10.0

NSC = 2
NTILE = 16
NWORK = NSC * NTILE
NPAD = 100352
SLICE1 = NPAD // NTILE
SLICE2 = NPAD // NWORK
EPT = E // NWORK
W = 2000
NWIN = EPT // W

_mesh = plsc.VectorSubcoreMesh(core_axis_name="c", subcore_axis_name="s")


@functools.partial(
    pl.kernel,
    out_type=jax.ShapeDtypeStruct((NSC * NPAD,), jnp.float32),
    mesh=_mesh,
    compiler_params=pltpu.CompilerParams(needs_layout_passes=False),
    scratch_types=[
        pltpu.VMEM((NPAD,), jnp.float32),
        [pltpu.VMEM((W,), jnp.int32)] * 2,
        [pltpu.VMEM((W,), jnp.int32)] * 2,
        [pltpu.VMEM((W,), jnp.float32)] * 2,
        [pltpu.VMEM((W,), jnp.float32)] * 2,
        pltpu.VMEM((SLICE1,), jnp.float32),
        pltpu.VMEM_SHARED((NPAD,), jnp.float32),
        [pltpu.SemaphoreType.DMA] * 2,
        [pltpu.SemaphoreType.DMA] * 2,
    ],
)
def _scatter_phase(v_hbm, src_hbm, dst_hbm, w_hbm, out_hbm,
                   vtab, srcb, dstb, wb, msgb, slicebuf, acc, lsem, ssem):
    cid = lax.axis_index("c")
    sid = lax.axis_index("s")
    gwid = cid * NTILE + sid
    nbase = sid * SLICE1
    ebase = gwid * EPT

    def _issue_loads(g, b):
        off = ebase + g * W
        pltpu.async_copy(src_hbm.at[pl.ds(off, W)], srcb[b], lsem[b])
        pltpu.async_copy(dst_hbm.at[pl.ds(off, W)], dstb[b], lsem[b])
        pltpu.async_copy(w_hbm.at[pl.ds(off, W)], wb[b], lsem[b])

    def _wait_loads(b):
        pltpu.make_async_copy(src_hbm.at[pl.ds(0, W)], srcb[b], lsem[b]).wait()
        pltpu.make_async_copy(dst_hbm.at[pl.ds(0, W)], dstb[b], lsem[b]).wait()
        pltpu.make_async_copy(w_hbm.at[pl.ds(0, W)], wb[b], lsem[b]).wait()

    def _issue_scatter(b):
        pltpu.async_copy(msgb[b], acc.at[dstb[b]], ssem[b], add=True)

    def _wait_scatter(b):
        pltpu.make_async_copy(msgb[b], acc.at[dstb[b]], ssem[b]).wait()

    def _zero(j, carry):
        slicebuf[pl.ds(j * 16, 16)] = jnp.zeros((16,), jnp.float32)
        return carry
    lax.fori_loop(0, SLICE1 // 16, _zero, 0)
    pltpu.sync_copy(slicebuf, acc.at[pl.ds(nbase, SLICE1)])

    pltpu.async_copy(v_hbm, vtab, lsem[0])
    _issue_loads(0, 0)
    pltpu.make_async_copy(v_hbm, vtab, lsem[0]).wait()
    plsc.subcore_barrier()

    def _compute(b):
        def body(j, carry):
            sl = pl.ds(j * 16, 16)
            idx = srcb[b][sl]
            x = plsc.load_gather(vtab, [idx])
            msgb[b][sl] = jnp.minimum(jnp.maximum(x, 0.0), CLAMP) * wb[b][sl]
            return carry
        lax.fori_loop(0, W // 16, body, 0, unroll=8)

    def _pair(k, carry):
        _wait_loads(0)
        _compute(0)

        @pl.when(k >= 1)
        def _():
            _wait_scatter(1)
        _issue_scatter(0)
        _issue_loads(2 * k + 1, 1)

        _wait_loads(1)
        _compute(1)
        _wait_scatter(0)
        _issue_scatter(1)

        @pl.when(k < NWIN // 2 - 1)
        def _():
            _issue_loads(2 * k + 2, 0)
        return carry

    lax.fori_loop(0, NWIN // 2, _pair, 0)
    _wait_scatter(1)
    plsc.subcore_barrier()

    pltpu.sync_copy(acc.at[pl.ds(nbase, SLICE1)], slicebuf)
    pltpu.sync_copy(slicebuf, out_hbm.at[pl.ds(cid * NPAD + nbase, SLICE1)])


@functools.partial(
    pl.kernel,
    out_type=jax.ShapeDtypeStruct((NPAD,), jnp.float32),
    mesh=_mesh,
    scratch_types=[
        pltpu.VMEM((SLICE2,), jnp.float32),
        pltpu.VMEM((SLICE2,), jnp.float32),
        pltpu.VMEM((SLICE2,), jnp.float32),
        pltpu.VMEM((SLICE2,), jnp.float32),
        pltpu.VMEM((SLICE2,), jnp.float32),
        pltpu.VMEM((SLICE2,), jnp.float32),
    ],
)
def _combine_phase(part_hbm, v_hbm, tau_hbm, stim_hbm, bias_hbm, out_hbm,
                   p0, p1, vb, tb, sb, bb):
    cid = lax.axis_index("c")
    sid = lax.axis_index("s")
    nb = (cid * NTILE + sid) * SLICE2
    pltpu.sync_copy(part_hbm.at[pl.ds(nb, SLICE2)], p0)
    pltpu.sync_copy(part_hbm.at[pl.ds(NPAD + nb, SLICE2)], p1)
    pltpu.sync_copy(v_hbm.at[pl.ds(nb, SLICE2)], vb)
    pltpu.sync_copy(tau_hbm.at[pl.ds(nb, SLICE2)], tb)
    pltpu.sync_copy(stim_hbm.at[pl.ds(nb, SLICE2)], sb)
    pltpu.sync_copy(bias_hbm.at[pl.ds(nb, SLICE2)], bb)

    def _compute(j, carry):
        sl = pl.ds(j * 16, 16)
        vc = jnp.minimum(jnp.maximum(vb[sl], -CLAMP), CLAMP)
        p0[sl] = (p0[sl] + p1[sl] + sb[sl] + bb[sl] - vc) / tb[sl]
        return carry
    lax.fori_loop(0, SLICE2 // 16, _compute, 0, unroll=8)
    pltpu.sync_copy(p0, out_hbm.at[pl.ds(nb, SLICE2)])


def kernel(t, v, edge_index, w_edge, tau, stimulus, bias):
    pad = NPAD - N
    vp = jnp.pad(v, (0, pad))
    taup = jnp.pad(tau, (0, pad), constant_values=1.0)
    stimp = jnp.pad(stimulus, (0, pad))
    biasp = jnp.pad(bias, (0, pad))
    src = edge_index[0]
    dst = edge_index[1]
    partial = _scatter_phase(vp, src, dst, w_edge)
    dvp = _combine_phase(partial, vp, taup, stimp, biasp)
    return dvp[:N]


# speedup vs baseline: 275.4873x; 1.3013x over previous
import functools

import jax
import jax.numpy as jnp
from jax import lax
from jax.experimental import pallas as pl
from jax.experimental.pallas import tpu as pltpu
from jax.experimental.pallas import tpu_sc as plsc

N = 100000
E = 6400000
CLAMP = 10.0

NSC = 2
NTILE = 16
NWORK = NSC * NTILE
NPAD = 100352
SLICE1 = NPAD // NTILE
SLICE2 = NPAD // NWORK
EPT = E // NWORK
W = 2000
NWIN = EPT // W

_mesh = plsc.VectorSubcoreMesh(core_axis_name="c", subcore_axis_name="s")


@functools.partial(
    pl.kernel,
    out_type=jax.ShapeDtypeStruct((NWORK * NPAD,), jnp.float32),
    mesh=_mesh,
    compiler_params=pltpu.CompilerParams(needs_layout_passes=False),
    scratch_types=[
        pltpu.VMEM((NPAD,), jnp.float32),
        [pltpu.VMEM((W,), jnp.int32)] * 2,
        [pltpu.VMEM((W,), jnp.int32)] * 2,
        [pltpu.VMEM((W,), jnp.float32)] * 2,
        [pltpu.VMEM((W,), jnp.float32)] * 2,
        pltpu.VMEM_SHARED((NPAD,), jnp.float32),
        [pltpu.SemaphoreType.DMA] * 2,
        [pltpu.SemaphoreType.DMA] * 2,
        pltpu.SemaphoreType.DMA,
    ],
)
def _scatter_phase(v_hbm, src_hbm, dst_hbm, w_hbm, out_hbm,
                   acc, srcb, dstb, wb, valb, vsh, lsem, gsem, psem):
    cid = lax.axis_index("c")
    sid = lax.axis_index("s")
    gwid = cid * NTILE + sid
    nbase = sid * SLICE1
    ebase = gwid * EPT

    def _issue_loads(g, b):
        off = ebase + g * W
        pltpu.async_copy(src_hbm.at[pl.ds(off, W)], srcb[b], lsem[b])
        pltpu.async_copy(dst_hbm.at[pl.ds(off, W)], dstb[b], lsem[b])
        pltpu.async_copy(w_hbm.at[pl.ds(off, W)], wb[b], lsem[b])

    def _wait_loads(b):
        pltpu.make_async_copy(src_hbm.at[pl.ds(0, W)], srcb[b], lsem[b]).wait()
        pltpu.make_async_copy(dst_hbm.at[pl.ds(0, W)], dstb[b], lsem[b]).wait()
        pltpu.make_async_copy(w_hbm.at[pl.ds(0, W)], wb[b], lsem[b]).wait()

    def _issue_gather(b):
        pltpu.async_copy(vsh.at[srcb[b]], valb[b], gsem[b])

    def _wait_gather(b):
        pltpu.make_async_copy(vsh.at[srcb[b]], valb[b], gsem[b]).wait()

    def _zero(j, carry):
        acc[pl.ds(j * 16, 16)] = jnp.zeros((16,), jnp.float32)
        return carry
    lax.fori_loop(0, NPAD // 16, _zero, 0, unroll=8)

    pltpu.sync_copy(v_hbm.at[pl.ds(nbase, SLICE1)],
                    vsh.at[pl.ds(nbase, SLICE1)])
    _issue_loads(0, 0)
    _wait_loads(0)
    plsc.subcore_barrier()
    _issue_gather(0)
    _issue_loads(1, 1)

    def _compute(b):
        def body(j, carry):
            sl = pl.ds(j * 16, 16)
            x = valb[b][sl]
            m = jnp.minimum(jnp.maximum(x, 0.0), CLAMP) * wb[b][sl]
            plsc.addupdate_scatter(acc, [dstb[b][sl]], m)
            return carry
        lax.fori_loop(0, W // 16, body, 0, unroll=8)

    def _pair(m, carry):
        _wait_gather(0)
        _compute(0)
        _wait_loads(1)
        _issue_gather(1)

        @pl.when(m < NWIN // 2 - 1)
        def _():
            _issue_loads(2 * m + 2, 0)

        _wait_gather(1)
        _compute(1)

        @pl.when(m < NWIN // 2 - 1)
        def _():
            _wait_loads(0)
            _issue_gather(0)
            _issue_loads(2 * m + 3, 1)
        return carry

    lax.fori_loop(0, NWIN // 2, _pair, 0)

    for j in range(NWORK):
        pltpu.async_copy(
            acc.at[pl.ds(j * SLICE2, SLICE2)],
            out_hbm.at[pl.ds(j * (NWORK * SLICE2) + gwid * SLICE2, SLICE2)],
            psem)
    for j in range(NWORK):
        pltpu.make_async_copy(
            acc.at[pl.ds(0, SLICE2)],
            out_hbm.at[pl.ds(0, SLICE2)],
            psem).wait()


@functools.partial(
    pl.kernel,
    out_type=jax.ShapeDtypeStruct((NPAD,), jnp.float32),
    mesh=_mesh,
    compiler_params=pltpu.CompilerParams(needs_layout_passes=False),
    scratch_types=[
        pltpu.VMEM((NWORK * SLICE2,), jnp.float32),
        pltpu.VMEM((SLICE2,), jnp.float32),
        pltpu.VMEM((SLICE2,), jnp.float32),
        pltpu.VMEM((SLICE2,), jnp.float32),
        pltpu.VMEM((SLICE2,), jnp.float32),
        pltpu.VMEM((SLICE2,), jnp.float32),
    ],
)
def _combine_phase(part_hbm, v_hbm, tau_hbm, stim_hbm, bias_hbm, out_hbm,
                   pbuf, vb, tb, sb, bb, ob):
    cid = lax.axis_index("c")
    sid = lax.axis_index("s")
    gwid = cid * NTILE + sid
    nb = gwid * SLICE2
    pltpu.sync_copy(part_hbm.at[pl.ds(gwid * (NWORK * SLICE2), NWORK * SLICE2)],
                    pbuf)
    pltpu.sync_copy(v_hbm.at[pl.ds(nb, SLICE2)], vb)
    pltpu.sync_copy(tau_hbm.at[pl.ds(nb, SLICE2)], tb)
    pltpu.sync_copy(stim_hbm.at[pl.ds(nb, SLICE2)], sb)
    pltpu.sync_copy(bias_hbm.at[pl.ds(nb, SLICE2)], bb)

    def _compute(j, carry):
        sl = pl.ds(j * 16, 16)
        s = pbuf[sl]
        for r in range(1, NWORK):
            s = s + pbuf[pl.ds(r * SLICE2 + j * 16, 16)]
        vc = jnp.minimum(jnp.maximum(vb[sl], -CLAMP), CLAMP)
        ob[sl] = (s + sb[sl] + bb[sl] - vc) / tb[sl]
        return carry
    lax.fori_loop(0, SLICE2 // 16, _compute, 0, unroll=2)
    pltpu.sync_copy(ob, out_hbm.at[pl.ds(nb, SLICE2)])


def kernel(t, v, edge_index, w_edge, tau, stimulus, bias):
    pad = NPAD - N
    vp = jnp.pad(v, (0, pad))
    taup = jnp.pad(tau, (0, pad), constant_values=1.0)
    stimp = jnp.pad(stimulus, (0, pad))
    biasp = jnp.pad(bias, (0, pad))
    src = edge_index[0]
    dst = edge_index[1]
    partial = _scatter_phase(vp, src, dst, w_edge)
    dvp = _combine_phase(partial, vp, taup, stimp, biasp)
    return dvp[:N]
